# baseline (speedup 1.0000x reference)
import jax
import jax.numpy as jnp
from jax.experimental import pallas as pl


def _sage_block_kernel(adj_ref, feat_ref, nf_ref, w_ref, out_ref):
    adj_blk = adj_ref[...]
    acc = jnp.dot(adj_blk, nf_ref[...],
                  preferred_element_type=jnp.float32)
    deg = jnp.sum(adj_blk, axis=1, keepdims=True) + 1.0
    neigh = acc / deg

    d = feat_ref.shape[1]
    w1 = w_ref[:, :d]
    w2 = w_ref[:, d:]
    dn = (((1,), (1,)), ((), ()))
    out_ref[...] = (
        jax.lax.dot_general(feat_ref[...], w1, dn,
                            preferred_element_type=jnp.float32)
        + jax.lax.dot_general(neigh, w2, dn,
                              preferred_element_type=jnp.float32)
    )


def _pick_block(n):
    for bm in (200, 80, 40, 16, 8):
        if n % bm == 0:
            return bm
    return n


def kernel(adj, features, neigh_feats, W):
    n, d = features.shape
    bm = _pick_block(n)
    return pl.pallas_call(
        _sage_block_kernel,
        grid=(n // bm,),
        in_specs=[
            pl.BlockSpec((bm, n), lambda i: (i, 0)),
            pl.BlockSpec((bm, d), lambda i: (i, 0)),
            pl.BlockSpec((n, d), lambda i: (0, 0)),
            pl.BlockSpec(W.shape, lambda i: (0, 0)),
        ],
        out_specs=pl.BlockSpec((bm, d), lambda i: (i, 0)),
        out_shape=jax.ShapeDtypeStruct((n, d), jnp.float32),
    )(adj, features, neigh_feats, W)


# speedup vs baseline: 1.0522x; 1.0522x over previous
import jax
import jax.numpy as jnp
from jax.experimental import pallas as pl


def _sage_block_kernel(adj_ref, feat_ref, nf_ref, w_ref, out_ref):
    adj_blk = adj_ref[...]
    acc = jnp.dot(adj_blk, nf_ref[...],
                  preferred_element_type=jnp.float32)
    deg = jnp.sum(adj_blk, axis=1, keepdims=True) + 1.0
    neigh = acc / deg

    d = feat_ref.shape[1]
    w1 = w_ref[:, :d]
    w2 = w_ref[:, d:]
    dn = (((1,), (1,)), ((), ()))
    out_ref[...] = (
        jax.lax.dot_general(feat_ref[...], w1, dn,
                            preferred_element_type=jnp.float32)
        + jax.lax.dot_general(neigh, w2, dn,
                              preferred_element_type=jnp.float32)
    )


def _pick_block(n):
    for bm in (400, 200, 80, 40, 16, 8):
        if n % bm == 0:
            return bm
    return n


def kernel(adj, features, neigh_feats, W):
    n, d = features.shape
    bm = _pick_block(n)
    return pl.pallas_call(
        _sage_block_kernel,
        grid=(n // bm,),
        in_specs=[
            pl.BlockSpec((bm, n), lambda i: (i, 0)),
            pl.BlockSpec((bm, d), lambda i: (i, 0)),
            pl.BlockSpec((n, d), lambda i: (0, 0)),
            pl.BlockSpec(W.shape, lambda i: (0, 0)),
        ],
        out_specs=pl.BlockSpec((bm, d), lambda i: (i, 0)),
        out_shape=jax.ShapeDtypeStruct((n, d), jnp.float32),
    )(adj, features, neigh_feats, W)
